# baseline (speedup 1.0000x reference)
import functools

import jax
import jax.numpy as jnp
from jax.experimental import pallas as pl
from jax.experimental.pallas import tpu as pltpu

_K = 16
_J = 6
_BIG = 2**30


def _knn_block_kernel(prow_ref, pcols_ref, out_ref, *, n, k):
    xi = prow_ref[:, 0:1]
    yi = prow_ref[:, 1:2]
    zi = prow_ref[:, 2:3]

    def distances():
        dx = xi - pcols_ref[0:1, :]
        dy = yi - pcols_ref[1:2, :]
        dz = zi - pcols_ref[2:3, :]
        return dx * dx + dy * dy + dz * dz

    d = distances()
    r = d.shape[0]
    n_chunks = n // 128

    d3 = d.reshape(r, n_chunks, 128)
    cbase = jax.lax.broadcasted_iota(jnp.int32, (r, n_chunks), 1) * 128
    vals, gidx = [], []
    for _ in range(_J):
        li = jnp.argmin(d3, axis=2).astype(jnp.int32)
        m = jnp.min(d3, axis=2)
        vals.append(m)
        gidx.append(cbase + li)
        d3 = jnp.where(jax.lax.broadcasted_iota(jnp.int32, d3.shape, 2)
                       == li[:, :, None], jnp.inf, d3)
    cand_v = jnp.concatenate(vals, axis=1)
    cand_i = jnp.concatenate(gidx, axis=1)

    cols = []
    m = None
    for _ in range(k):
        m = jnp.min(cand_v, axis=1, keepdims=True)
        idx = jnp.min(jnp.where(cand_v == m, cand_i, _BIG), axis=1,
                      keepdims=True)
        cols.append(idx)
        cand_v = jnp.where(cand_i == idx, jnp.inf, cand_v)
    out_ref[:, :] = jnp.concatenate(cols, axis=1)

    bad = jnp.any(vals[-1] <= m)

    @pl.when(bad)
    def _fallback():
        dd = distances()
        iota = jax.lax.broadcasted_iota(jnp.int32, dd.shape, 1)
        fcols = []
        for _ in range(k):
            fidx = jnp.argmin(dd, axis=1).astype(jnp.int32)[:, None]
            fcols.append(fidx)
            dd = jnp.where(iota == fidx, jnp.inf, dd)
        out_ref[:, :] = jnp.concatenate(fcols, axis=1)


@jax.jit
def kernel(p, active_nodes):
    n = p.shape[0]
    block_r = 128
    pcols = jnp.zeros((8, n), dtype=p.dtype).at[:3, :].set(p.T)
    idxs = pl.pallas_call(
        functools.partial(_knn_block_kernel, n=n, k=_K),
        grid=(n // block_r,),
        in_specs=[
            pl.BlockSpec((block_r, 3), lambda i: (i, 0)),
            pl.BlockSpec((8, n), lambda i: (0, 0)),
        ],
        out_specs=pl.BlockSpec((block_r, _K), lambda i: (i, 0)),
        out_shape=jax.ShapeDtypeStruct((n, _K), jnp.int32),
        compiler_params=pltpu.CompilerParams(
            dimension_semantics=("arbitrary",),
        ),
    )(p, pcols)
    row = jnp.broadcast_to(jnp.arange(n, dtype=idxs.dtype)[:, None], (n, _K))
    s = jnp.where(active_nodes[:, None], idxs, n - 1)
    r = jnp.where(active_nodes[:, None], row, n - 1)
    return s.reshape(-1), r.reshape(-1)


# speedup vs baseline: 1.1079x; 1.1079x over previous
import functools

import jax
import jax.numpy as jnp
from jax.experimental import pallas as pl
from jax.experimental.pallas import tpu as pltpu

_K = 16
_J = 6
_BIG = 2**30
_BLOCK_R = 128
_CHUNK = 64


def _knn_block_kernel(pj_ref, pcols_ref, out_ref, *, n, k):
    xi = pcols_ref[0:1, :]
    yi = pcols_ref[1:2, :]
    zi = pcols_ref[2:3, :]

    def distances():
        dx = pj_ref[:, 0:1] - xi
        dy = pj_ref[:, 1:2] - yi
        dz = pj_ref[:, 2:3] - zi
        return dx * dx + dy * dy + dz * dz

    t = distances()
    r = t.shape[1]
    n_chunks = n // _CHUNK

    t3 = t.reshape(n_chunks, _CHUNK, r)
    cbase = jax.lax.broadcasted_iota(jnp.int32, (n_chunks, 1, r), 0) * _CHUNK
    vals, gidx = [], []
    for _ in range(_J):
        li = jnp.argmin(t3, axis=1, keepdims=True).astype(jnp.int32)
        m = jnp.min(t3, axis=1, keepdims=True)
        vals.append(m)
        gidx.append(cbase + li)
        t3 = jnp.where(jax.lax.broadcasted_iota(jnp.int32, t3.shape, 1)
                       == li, jnp.inf, t3)
    cand_v = jnp.concatenate(vals, axis=1).reshape(n_chunks * _J, r)
    cand_i = jnp.concatenate(gidx, axis=1).reshape(n_chunks * _J, r)

    rows = []
    m = None
    for _ in range(k):
        m = jnp.min(cand_v, axis=0, keepdims=True)
        idx = jnp.min(jnp.where(cand_v == m, cand_i, _BIG), axis=0,
                      keepdims=True)
        rows.append(idx)
        cand_v = jnp.where(cand_i == idx, jnp.inf, cand_v)
    out_ref[:, :] = jnp.concatenate(rows, axis=0)

    bad = jnp.any(vals[-1].reshape(n_chunks, r) <= m)

    @pl.when(bad)
    def _fallback():
        tt = distances()
        iota = jax.lax.broadcasted_iota(jnp.int32, tt.shape, 0)
        frows = []
        for _ in range(k):
            fm = jnp.min(tt, axis=0, keepdims=True)
            fidx = jnp.min(jnp.where(tt == fm, iota, _BIG), axis=0,
                           keepdims=True)
            frows.append(fidx)
            tt = jnp.where(iota == fidx, jnp.inf, tt)
        out_ref[:, :] = jnp.concatenate(frows, axis=0)


@jax.jit
def kernel(p, active_nodes):
    n = p.shape[0]
    pcols = jnp.zeros((8, n), dtype=p.dtype).at[:3, :].set(p.T)
    idxs_t = pl.pallas_call(
        functools.partial(_knn_block_kernel, n=n, k=_K),
        grid=(n // _BLOCK_R,),
        in_specs=[
            pl.BlockSpec((n, 3), lambda i: (0, 0)),
            pl.BlockSpec((8, _BLOCK_R), lambda i: (0, i)),
        ],
        out_specs=pl.BlockSpec((_K, _BLOCK_R), lambda i: (0, i)),
        out_shape=jax.ShapeDtypeStruct((_K, n), jnp.int32),
        compiler_params=pltpu.CompilerParams(
            dimension_semantics=("arbitrary",),
        ),
    )(p, pcols)
    idxs = idxs_t.T
    row = jnp.broadcast_to(jnp.arange(n, dtype=idxs.dtype)[:, None], (n, _K))
    s = jnp.where(active_nodes[:, None], idxs, n - 1)
    r = jnp.where(active_nodes[:, None], row, n - 1)
    return s.reshape(-1), r.reshape(-1)


# speedup vs baseline: 1.8619x; 1.6805x over previous
import functools

import jax
import jax.numpy as jnp
from jax.experimental import pallas as pl
from jax.experimental.pallas import tpu as pltpu

_K = 16
_J = 5
_BIG = 2**30
_BLOCK_R = 128
_W = 64


def _knn_block_kernel(pj_ref, pcols_ref, out_ref, *, n, k):
    xi = pcols_ref[0:1, :]
    yi = pcols_ref[1:2, :]
    zi = pcols_ref[2:3, :]
    csz = n // _W

    def dist_slice(w):
        dx = pj_ref[pl.ds(w * csz, csz), 0:1] - xi
        dy = pj_ref[pl.ds(w * csz, csz), 1:2] - yi
        dz = pj_ref[pl.ds(w * csz, csz), 2:3] - zi
        return dx * dx + dy * dy + dz * dz

    t = [dist_slice(w) for w in range(_W)]

    def tournament(slices):
        cur = []
        for a in range(0, _W, 2):
            c = slices[a + 1] < slices[a]
            cur.append((jnp.where(c, slices[a + 1], slices[a]),
                        jnp.where(c, a + 1, a)))
        while len(cur) > 1:
            nxt = []
            for a in range(0, len(cur), 2):
                v0, w0 = cur[a]
                v1, w1 = cur[a + 1]
                c = v1 < v0
                nxt.append((jnp.where(c, v1, v0), jnp.where(c, w1, w0)))
            cur = nxt
        return cur[0]

    si = jax.lax.broadcasted_iota(jnp.int32, (csz, _BLOCK_R), 0)
    vals, gidx = [], []
    for _ in range(_J):
        m, wdep = tournament(t)
        vals.append(m)
        gidx.append(wdep * csz + si)
        t = [jnp.where(wdep == w, jnp.inf, t[w]) for w in range(_W)]
    cand_v = jnp.concatenate(vals, axis=0)
    cand_i = jnp.concatenate(gidx, axis=0)

    rows = []
    m = None
    for _ in range(k):
        m = jnp.min(cand_v, axis=0, keepdims=True)
        idx = jnp.min(jnp.where(cand_v == m, cand_i, _BIG), axis=0,
                      keepdims=True)
        rows.append(idx)
        cand_v = jnp.where(cand_i == idx, jnp.inf, cand_v)
    out_ref[:, :] = jnp.concatenate(rows, axis=0)

    bad = jnp.any(vals[-1] <= m)

    @pl.when(bad)
    def _fallback():
        tt = jnp.concatenate([dist_slice(w) for w in range(_W)], axis=0)
        iota = jax.lax.broadcasted_iota(jnp.int32, tt.shape, 0)
        frows = []
        for _ in range(k):
            fm = jnp.min(tt, axis=0, keepdims=True)
            fidx = jnp.min(jnp.where(tt == fm, iota, _BIG), axis=0,
                           keepdims=True)
            frows.append(fidx)
            tt = jnp.where(iota == fidx, jnp.inf, tt)
        out_ref[:, :] = jnp.concatenate(frows, axis=0)


@jax.jit
def kernel(p, active_nodes):
    n = p.shape[0]
    pcols = jnp.zeros((8, n), dtype=p.dtype).at[:3, :].set(p.T)
    idxs_t = pl.pallas_call(
        functools.partial(_knn_block_kernel, n=n, k=_K),
        grid=(n // _BLOCK_R,),
        in_specs=[
            pl.BlockSpec((n, 3), lambda i: (0, 0)),
            pl.BlockSpec((8, _BLOCK_R), lambda i: (0, i)),
        ],
        out_specs=pl.BlockSpec((_K, _BLOCK_R), lambda i: (0, i)),
        out_shape=jax.ShapeDtypeStruct((_K, n), jnp.int32),
        compiler_params=pltpu.CompilerParams(
            dimension_semantics=("arbitrary",),
        ),
    )(p, pcols)
    idxs = idxs_t.T
    row = jnp.broadcast_to(jnp.arange(n, dtype=idxs.dtype)[:, None], (n, _K))
    s = jnp.where(active_nodes[:, None], idxs, n - 1)
    r = jnp.where(active_nodes[:, None], row, n - 1)
    return s.reshape(-1), r.reshape(-1)
